# baseline (speedup 1.0000x reference)
import jax
import jax.numpy as jnp
from jax.experimental import pallas as pl
from jax.experimental.pallas import tpu as pltpu

_STRIDE = 32
_NDMA = 8


def kernel(x):
    b, h, s, d = x.shape
    k = s // _STRIDE
    n = b * h
    x2 = x.reshape(n, k, _STRIDE * d)
    chunk = n // _NDMA

    def body(in_ref, out_ref, *sems):
        for i in range(_NDMA):
            pltpu.make_async_copy(
                in_ref.at[pl.ds(i * chunk, chunk), :, pl.ds(0, d)],
                out_ref.at[pl.ds(i * chunk, chunk)],
                sems[i],
            ).start()
        for i in range(_NDMA):
            pltpu.make_async_copy(
                in_ref.at[pl.ds(i * chunk, chunk), :, pl.ds(0, d)],
                out_ref.at[pl.ds(i * chunk, chunk)],
                sems[i],
            ).wait()

    out = pl.pallas_call(
        body,
        in_specs=[pl.BlockSpec(memory_space=pl.ANY)],
        out_specs=pl.BlockSpec(memory_space=pl.ANY),
        out_shape=jax.ShapeDtypeStruct((n, k, d), x.dtype),
        scratch_shapes=[pltpu.SemaphoreType.DMA] * _NDMA,
    )(x2)
    return out.reshape(b, h, k, d)
